# baseline (speedup 1.0000x reference)
import jax
import jax.numpy as jnp
import numpy as np
from jax import lax
from jax.experimental import pallas as pl
from jax.experimental.pallas import tpu as pltpu
from jax.experimental.pallas import tpu_sc as plsc

D = 64
LANES = 16
NC, NS = 2, 16
NW = NC * NS
BBLK = 128
DT = D // 8


def _make_kernel(n_l, n_b):
    assert n_b == NW * BBLK and n_l % 2 == 0
    mesh = plsc.VectorSubcoreMesh(core_axis_name="c", subcore_axis_name="s",
                                  num_cores=NC, num_subcores=NS)
    def body(seq_hbm, t_hbm, r_hbm, tok_hbm, tim_hbm, rat_hbm,
             x_out, tim_out,
             timtbl, rattbl, sidx, tidx, ridx, rows, xblk, timblk,
             gsem, isem, wsem):
        w = lax.axis_index("s") * NC + lax.axis_index("c")
        col0 = w * BBLK

        pltpu.sync_copy(tim_hbm, timtbl)
        pltpu.sync_copy(rat_hbm, rattbl)

        def fire_idx(l, b):
            pltpu.async_copy(seq_hbm.at[l, pl.ds(col0, BBLK)], sidx.at[b],
                             isem.at[b])
            pltpu.async_copy(t_hbm.at[l, pl.ds(col0, BBLK)], tidx.at[b],
                             isem.at[b])
            pltpu.async_copy(r_hbm.at[l, pl.ds(col0, BBLK)], ridx.at[b],
                             isem.at[b])

        def wait_idx(b):
            for ref in (sidx, tidx, ridx):
                pltpu.make_async_copy(seq_hbm.at[0, pl.ds(0, BBLK)],
                                      ref.at[b], isem.at[b]).wait()

        def fire_gather(b):
            pltpu.async_copy(tok_hbm.at[sidx.at[b]], rows.at[b], gsem.at[b])

        def wait_gather(b):
            pltpu.make_async_copy(tok_hbm.at[sidx.at[b]], rows.at[b],
                                  gsem.at[b]).wait()

        def fire_wb(l, b):
            pltpu.async_copy(xblk.at[b], x_out.at[l, :, pl.ds(w, 1)],
                             wsem.at[b])
            pltpu.async_copy(timblk.at[b], tim_out.at[l, :, pl.ds(w, 1)],
                             wsem.at[b])

        def wait_wb(b):
            pltpu.make_async_copy(xblk.at[b], x_out.at[0, :, pl.ds(0, 1)],
                                  wsem.at[b]).wait()
            pltpu.make_async_copy(timblk.at[b], tim_out.at[0, :, pl.ds(0, 1)],
                                  wsem.at[b]).wait()

        def compute(b):
            iota16 = lax.iota(jnp.int32, LANES)
            bv = jnp.full((LANES,), b, jnp.int32)

            @plsc.parallel_loop(0, BBLK, step=LANES)
            def _(jb):
                t16 = tidx[b, pl.ds(jb, LANES)]
                r16 = ridx[b, pl.ds(jb, LANES)]
                tbase = t16 * D
                rbase = r16 * D
                rv = iota16 + jb
                for dt in range(DT):
                    for d8 in range(8):
                        d = dt * 8 + d8
                        colv = iota16 * 0 + d
                        tokv = plsc.load_gather(rows, [bv, rv, colv])
                        timv = plsc.load_gather(timtbl, [tbase + d])
                        ratv = plsc.load_gather(rattbl, [rbase + d])
                        sl = pl.ds(jb, LANES)
                        timblk[b, dt, 0, d8, sl] = timv
                        xblk[b, dt, 0, d8, sl] = tokv + timv + ratv

        fire_idx(0, 0)
        fire_idx(1, 1)
        wait_idx(0)
        fire_gather(0)

        def pair_body(g, carry):
            for b in range(2):
                l = g * 2 + b
                nb = 1 - b

                @pl.when(l >= 1)
                def _():
                    wait_wb(nb)

                @pl.when(l + 1 < n_l)
                def _():
                    wait_idx(nb)
                    fire_gather(nb)

                wait_gather(b)
                compute(b)

                @pl.when(l + 2 < n_l)
                def _():
                    fire_idx(l + 2, b)

                fire_wb(l, b)
            return carry
        lax.fori_loop(0, n_l // 2, pair_body, 0)
        wait_wb((n_l - 1) % 2)

    return pl.kernel(
        body,
        out_type=(jax.ShapeDtypeStruct((n_l, DT, NW, 8, BBLK), jnp.float32),
                  jax.ShapeDtypeStruct((n_l, DT, NW, 8, BBLK), jnp.float32)),
        mesh=mesh,
        scratch_types=[
            pltpu.VMEM((512 * D,), jnp.float32),
            pltpu.VMEM((10 * D,), jnp.float32),
            pltpu.VMEM((2, BBLK), jnp.int32),
            pltpu.VMEM((2, BBLK), jnp.int32),
            pltpu.VMEM((2, BBLK), jnp.int32),
            pltpu.VMEM((2, BBLK, D), jnp.float32),
            pltpu.VMEM((2, DT, 1, 8, BBLK), jnp.float32),
            pltpu.VMEM((2, DT, 1, 8, BBLK), jnp.float32),
            pltpu.SemaphoreType.DMA((2,)),
            pltpu.SemaphoreType.DMA((2,)),
            pltpu.SemaphoreType.DMA((2,)),
        ],
        compiler_params=pltpu.CompilerParams(use_tc_tiling_on_sc=False,
                                             needs_layout_passes=False),
    )


def kernel(sequence, r, t, tok_table, rat_table, tim_table):
    B_, L_ = sequence.shape
    seqT = sequence.T.astype(jnp.int32)
    tT = t.T.astype(jnp.int32)
    rT = r.T.astype(jnp.int32)
    k = _make_kernel(L_, B_)
    x5, tim5 = k(seqT, tT, rT, tok_table,
                 tim_table.reshape(-1), rat_table.reshape(-1))
    x = x5.transpose(2, 4, 0, 1, 3).reshape(B_, L_, D)
    tim = tim5.transpose(2, 4, 0, 1, 3).reshape(B_, L_, D)
    return x, tim


# speedup vs baseline: 2.3796x; 2.3796x over previous
import jax
import jax.numpy as jnp
import numpy as np
from jax import lax
from jax.experimental import pallas as pl
from jax.experimental.pallas import tpu as pltpu
from jax.experimental.pallas import tpu_sc as plsc

D = 64
LANES = 16
NC, NS = 2, 16
NW = NC * NS
BBLK = 128
DT = D // 8


def _make_kernel(n_l, n_b):
    assert n_b == NW * BBLK and n_l % 2 == 0
    mesh = plsc.VectorSubcoreMesh(core_axis_name="c", subcore_axis_name="s",
                                  num_cores=NC, num_subcores=NS)
    def body(seq_hbm, t_hbm, r_hbm, tok_hbm, tim_hbm, rat_hbm,
             x_out, tim_out,
             timtbl, rattbl, sidx, tidx, ridx, rows, xblk, timblk,
             gsem, isem, wsem):
        w = lax.axis_index("s") * NC + lax.axis_index("c")
        col0 = w * BBLK

        pltpu.sync_copy(tim_hbm, timtbl)
        pltpu.sync_copy(rat_hbm, rattbl)

        def fire_idx(l, b):
            pltpu.async_copy(seq_hbm.at[l, pl.ds(col0, BBLK)], sidx.at[b],
                             isem.at[b])
            pltpu.async_copy(t_hbm.at[l, pl.ds(col0, BBLK)], tidx.at[b],
                             isem.at[b])
            pltpu.async_copy(r_hbm.at[l, pl.ds(col0, BBLK)], ridx.at[b],
                             isem.at[b])

        def wait_idx(b):
            for ref in (sidx, tidx, ridx):
                pltpu.make_async_copy(seq_hbm.at[0, pl.ds(0, BBLK)],
                                      ref.at[b], isem.at[b]).wait()

        def fire_gather(b):
            pltpu.async_copy(tok_hbm.at[sidx.at[b]], rows.at[b], gsem.at[b])

        def wait_gather(b):
            pltpu.make_async_copy(tok_hbm.at[sidx.at[b]], rows.at[b],
                                  gsem.at[b]).wait()

        def fire_wb(l, b):
            pltpu.async_copy(xblk.at[b, :, :, :, pl.ds(0, BBLK)],
                             x_out.at[l, :, pl.ds(w, 1)], wsem.at[b])
            pltpu.async_copy(timblk.at[b, :, :, :, pl.ds(0, BBLK)],
                             tim_out.at[l, :, pl.ds(w, 1)], wsem.at[b])

        def wait_wb(b):
            pltpu.make_async_copy(xblk.at[b, :, :, :, pl.ds(0, BBLK)],
                                  x_out.at[0, :, pl.ds(0, 1)],
                                  wsem.at[b]).wait()
            pltpu.make_async_copy(timblk.at[b, :, :, :, pl.ds(0, BBLK)],
                                  tim_out.at[0, :, pl.ds(0, 1)],
                                  wsem.at[b]).wait()

        def compute(b):
            iota16 = lax.iota(jnp.int32, LANES)
            bcon = jnp.full((LANES,), b, jnp.int32)

            @plsc.parallel_loop(0, BBLK, step=LANES)
            def _(jb):
                t16 = tidx[b, pl.ds(jb, LANES)]
                r16 = ridx[b, pl.ds(jb, LANES)]
                for m in range(LANES):
                    btok = jb + m
                    tb = t16[m] * D
                    rb = r16[m] * D
                    colv = jnp.full((LANES,), btok, jnp.int32)
                    for g in range(D // LANES):
                        dvec = iota16 + (g * LANES)
                        dtv = dvec // 8
                        d8v = dvec % 8
                        tokv = rows[b, btok, pl.ds(g * LANES, LANES)]
                        timv = timtbl[pl.ds(tb + g * LANES, LANES)]
                        ratv = rattbl[pl.ds(rb + g * LANES, LANES)]
                        zcon = jnp.zeros((LANES,), jnp.int32)
                        idx = [bcon, dtv, zcon, d8v, colv]
                        plsc.store_scatter(timblk, idx, timv)
                        plsc.store_scatter(xblk, idx, tokv + timv + ratv)

        fire_idx(0, 0)
        fire_idx(1, 1)
        wait_idx(0)
        fire_gather(0)

        def pair_body(g, carry):
            for b in range(2):
                l = g * 2 + b
                nb = 1 - b

                @pl.when(l >= 1)
                def _():
                    wait_wb(nb)

                @pl.when(l + 1 < n_l)
                def _():
                    wait_idx(nb)
                    fire_gather(nb)

                wait_gather(b)
                compute(b)

                @pl.when(l + 2 < n_l)
                def _():
                    fire_idx(l + 2, b)

                fire_wb(l, b)
            return carry
        lax.fori_loop(0, n_l // 2, pair_body, 0)
        wait_wb((n_l - 1) % 2)

    return pl.kernel(
        body,
        out_type=(jax.ShapeDtypeStruct((n_l, DT, NW, 8, BBLK), jnp.float32),
                  jax.ShapeDtypeStruct((n_l, DT, NW, 8, BBLK), jnp.float32)),
        mesh=mesh,
        scratch_types=[
            pltpu.VMEM((512 * D,), jnp.float32),
            pltpu.VMEM((10 * D,), jnp.float32),
            pltpu.VMEM((2, BBLK), jnp.int32),
            pltpu.VMEM((2, BBLK), jnp.int32),
            pltpu.VMEM((2, BBLK), jnp.int32),
            pltpu.VMEM((2, BBLK, D), jnp.float32),
            pltpu.VMEM((2, DT, 1, 8, BBLK + 1), jnp.float32),
            pltpu.VMEM((2, DT, 1, 8, BBLK + 1), jnp.float32),
            pltpu.SemaphoreType.DMA((2,)),
            pltpu.SemaphoreType.DMA((2,)),
            pltpu.SemaphoreType.DMA((2,)),
        ],
        compiler_params=pltpu.CompilerParams(use_tc_tiling_on_sc=False,
                                             needs_layout_passes=False),
    )


def kernel(sequence, r, t, tok_table, rat_table, tim_table):
    B_, L_ = sequence.shape
    seqT = sequence.T.astype(jnp.int32)
    tT = t.T.astype(jnp.int32)
    rT = r.T.astype(jnp.int32)
    k = _make_kernel(L_, B_)
    x5, tim5 = k(seqT, tT, rT, tok_table,
                 tim_table.reshape(-1), rat_table.reshape(-1))
    x = x5.transpose(2, 4, 0, 1, 3).reshape(B_, L_, D)
    tim = tim5.transpose(2, 4, 0, 1, 3).reshape(B_, L_, D)
    return x, tim
